# baseline (speedup 1.0000x reference)
import functools

import jax
import jax.numpy as jnp
from jax import lax
from jax.experimental import pallas as pl
from jax.experimental.pallas import tpu as pltpu
from jax.experimental.pallas import tpu_sc as plsc

_NC = 2
_NS = 16
_NW = _NC * _NS
_L = 16


def _sc_mesh():
    return plsc.VectorSubcoreMesh(
        core_axis_name="c", subcore_axis_name="s",
        num_cores=_NC, num_subcores=_NS)


def _wid():
    return lax.axis_index("s") * _NC + lax.axis_index("c")


def _zero_ref(ref):
    zeros = jnp.zeros((_L,), jnp.float32)

    def body(i, carry):
        ref[pl.ds(i * _L, _L)] = zeros
        return carry

    lax.fori_loop(0, ref.shape[0] // _L, body, 0, unroll=4)


def _fast_rsqrt(x):
    i = plsc.bitcast(x, jnp.int32)
    i = jnp.int32(0x5F3759DF) - (i >> 1)
    y = plsc.bitcast(i, jnp.float32)
    for _ in range(3):
        y = y * (1.5 - 0.5 * x * y * y)
    return y


def _deg_body(dst_hbm, out_hbm, dst_v, acc_v):
    epw = dst_v.shape[0]
    wid = _wid()
    pltpu.sync_copy(dst_hbm.at[pl.ds(wid * epw, epw)], dst_v)
    _zero_ref(acc_v)
    ones = jnp.ones((_L,), jnp.float32)

    def body(i, carry):
        d = dst_v[pl.ds(i * _L, _L)]
        plsc.addupdate_scatter(acc_v, [d], ones)
        return carry

    lax.fori_loop(0, epw // _L, body, 0, unroll=4)
    pltpu.sync_copy(acc_v, out_hbm.at[wid])


def _deg_call(dst, npad):
    e = dst.shape[0]
    epw = e // _NW
    fn = pl.kernel(
        _deg_body,
        out_type=jax.ShapeDtypeStruct((_NW, npad), jnp.float32),
        mesh=_sc_mesh(),
        compiler_params=pltpu.CompilerParams(needs_layout_passes=False),
        scratch_types=[
            pltpu.VMEM((epw,), jnp.int32),
            pltpu.VMEM((npad,), jnp.float32),
        ],
    )
    return fn(dst)


def _msg_body(src_hbm, dst_hbm, degp_hbm, h_hbm, out_hbm,
              src_v, dst_v, g_v, acc_v, dp_v, hseg_v, gseg_v, shared_g):
    epw = src_v.shape[0]
    npad = g_v.shape[0]
    seg = npad // _NS
    sid = lax.axis_index("s")
    wid = _wid()

    pltpu.sync_copy(src_hbm.at[pl.ds(wid * epw, epw)], src_v)
    pltpu.sync_copy(dst_hbm.at[pl.ds(wid * epw, epw)], dst_v)
    pltpu.sync_copy(degp_hbm.at[:, pl.ds(sid * seg, seg)], dp_v)
    pltpu.sync_copy(h_hbm.at[pl.ds(sid * seg, seg)], hseg_v)

    def seg_body(v, carry):
        tot = jnp.ones((_L,), jnp.float32)
        for r in range(_NW):
            tot = tot + dp_v[r, pl.ds(v * _L, _L)]
        dis = _fast_rsqrt(tot)
        gseg_v[pl.ds(v * _L, _L)] = dis * hseg_v[pl.ds(v * _L, _L)]
        return carry

    lax.fori_loop(0, seg // _L, seg_body, 0, unroll=2)
    pltpu.sync_copy(gseg_v, shared_g.at[pl.ds(sid * seg, seg)])
    plsc.subcore_barrier()
    pltpu.sync_copy(shared_g, g_v)

    _zero_ref(acc_v)

    def body(i, carry):
        s = src_v[pl.ds(i * _L, _L)]
        d = dst_v[pl.ds(i * _L, _L)]
        vals = plsc.load_gather(g_v, [s])
        plsc.addupdate_scatter(acc_v, [d], vals)
        return carry

    lax.fori_loop(0, epw // _L, body, 0, unroll=4)
    pltpu.sync_copy(acc_v, out_hbm.at[wid])


def _msg_call(src, dst, degp, h, npad):
    e = src.shape[0]
    epw = e // _NW
    seg = npad // _NS
    fn = pl.kernel(
        _msg_body,
        out_type=jax.ShapeDtypeStruct((_NW, npad), jnp.float32),
        mesh=_sc_mesh(),
        compiler_params=pltpu.CompilerParams(needs_layout_passes=False),
        scratch_types=[
            pltpu.VMEM((epw,), jnp.int32),
            pltpu.VMEM((epw,), jnp.int32),
            pltpu.VMEM((npad,), jnp.float32),
            pltpu.VMEM((npad,), jnp.float32),
            pltpu.VMEM((_NW, seg), jnp.float32),
            pltpu.VMEM((seg,), jnp.float32),
            pltpu.VMEM((seg,), jnp.float32),
            pltpu.VMEM_SHARED((npad,), jnp.float32),
        ],
    )
    return fn(src, dst, degp, h)


def _mv_body(x_ref, w_ref, h_ref):
    h_ref[...] = lax.dot_general(
        w_ref[...], x_ref[...], (((1,), (1,)), ((), ())),
        preferred_element_type=jnp.float32)


def _mv_call(x, w_row):
    n = x.shape[0]
    return pl.pallas_call(
        _mv_body,
        out_shape=jax.ShapeDtypeStruct((1, n), jnp.float32),
    )(x, w_row)


def _fin_body(accp_ref, degp_ref, h_ref, b_ref, out_ref):
    n = out_ref.shape[1]
    tot = jnp.sum(accp_ref[:, :n], axis=0, keepdims=True)
    deg = jnp.sum(degp_ref[:, :n], axis=0, keepdims=True) + 1.0
    dis = lax.rsqrt(deg)
    g = dis * h_ref[...]
    z = dis * (tot + g) + b_ref[0, 0]
    out_ref[...] = jnp.maximum(z, 0.0) + jnp.log1p(jnp.exp(-jnp.abs(z)))


def _fin_call(accp, degp, h_row, b):
    n = h_row.shape[1]
    return pl.pallas_call(
        _fin_body,
        out_shape=jax.ShapeDtypeStruct((1, n), jnp.float32),
    )(accp, degp, h_row, b.reshape(1, 1))


@jax.jit
def kernel(x, edge_index, W, b):
    n, d = x.shape
    npad = -(-n // (_NS * _L)) * (_NS * _L)
    src = edge_index[0]
    dst = edge_index[1]
    h_row = _mv_call(x, W.reshape(1, d))
    degp = _deg_call(dst, npad)
    h_pad = jnp.zeros((npad,), jnp.float32).at[:n].set(h_row.reshape(n))
    accp = _msg_call(src, dst, degp, h_pad, npad)
    out_row = _fin_call(accp, degp, h_row, b)
    return out_row.reshape(n, 1)
